# baseline (speedup 1.0000x reference)
import jax
import jax.numpy as jnp
from jax import lax
from jax.experimental import pallas as pl
from jax.experimental.pallas import tpu as pltpu
from jax.experimental.pallas import tpu_sc as plsc

B = 1024
W = 200
NUM_INPUTS = 8
D = 64
HIST = 150
FUT = W - HIST

NC = 2
NS = 16
NW = NC * NS

CH_H, NCH_H = 120, 40
CH_F, NCH_F = 80, 20
S_PER_W = B // NW


def _sc_gather_e1(e1hi_hbm, e1lo_hbm, idxh_hbm, idxf_hbm, gh_hbm, gf_hbm,
                  idxh_v, idxf_v, r_a, r_b, sem_a, sem_b):
    wid = lax.axis_index("s") * NC + lax.axis_index("c")

    pltpu.sync_copy(idxh_hbm.at[wid], idxh_v)
    base_h = wid * (NCH_H * CH_H)

    def body_h(k, carry):
        j0 = 2 * k
        j1 = j0 + 1
        c0 = pltpu.async_copy(e1hi_hbm.at[idxh_v.at[j0]], r_a, sem_a)
        c1 = pltpu.async_copy(e1hi_hbm.at[idxh_v.at[j1]], r_b, sem_b)
        c0.wait()
        pltpu.sync_copy(r_a, gh_hbm.at[pl.ds(base_h + j0 * CH_H, CH_H)])
        c1.wait()
        pltpu.sync_copy(r_b, gh_hbm.at[pl.ds(base_h + j1 * CH_H, CH_H)])
        return carry

    lax.fori_loop(0, NCH_H // 2, body_h, 0)

    pltpu.sync_copy(idxf_hbm.at[wid], idxf_v)
    base_f = wid * (NCH_F * CH_F)

    def body_f(k, carry):
        j0 = 2 * k
        j1 = j0 + 1
        ra = r_a.at[pl.ds(0, CH_F)]
        rb = r_b.at[pl.ds(0, CH_F)]
        c0 = pltpu.async_copy(e1lo_hbm.at[idxf_v.at[j0]], ra, sem_a)
        c1 = pltpu.async_copy(e1lo_hbm.at[idxf_v.at[j1]], rb, sem_b)
        c0.wait()
        pltpu.sync_copy(ra, gf_hbm.at[pl.ds(base_f + j0 * CH_F, CH_F)])
        c1.wait()
        pltpu.sync_copy(rb, gf_hbm.at[pl.ds(base_f + j1 * CH_F, CH_F)])
        return carry

    lax.fori_loop(0, NCH_F // 2, body_f, 0)


def _sc_gather_e0(e0_hbm, idx0_hbm, s_hbm, idx0_v, r0_v, sem0):
    wid = lax.axis_index("s") * NC + lax.axis_index("c")
    pltpu.sync_copy(idx0_hbm.at[wid], idx0_v)
    pltpu.async_copy(e0_hbm.at[idx0_v.at[0]], r0_v, sem0).wait()
    pltpu.sync_copy(r0_v, s_hbm.at[pl.ds(wid * S_PER_W, S_PER_W)])


def _tc_body(xh_ref, xf_ref, gh_ref, gf_ref, wh_ref, bh_ref, wf_ref, bf_ref,
             hist_ref, fut_ref):
    gh = gh_ref[...]
    gf = gf_ref[...]
    low = lax.broadcasted_iota(jnp.int32, (1, 2 * D), 1) < D
    xh = xh_ref[...]
    xf = xf_ref[...]

    def d(x, i, wref, bref):
        return x[:, i:i + 1] * wref[i][None, :] + bref[i][None, :]

    hist_ref[:, 0:128] = jnp.where(low, d(xh, 7, wh_ref, bh_ref), gh)
    hist_ref[:, 128:256] = d(xh, 5, wh_ref, bh_ref) + d(xh, 6, wh_ref, bh_ref)
    hist_ref[:, 256:384] = d(xh, 2, wh_ref, bh_ref) + d(xh, 3, wh_ref, bh_ref)
    hist_ref[:, 384:448] = d(xh, 4, wh_ref, bh_ref)[:, 0:64]
    fut_ref[:, 0:128] = jnp.where(low, gf, d(xf, 5, wf_ref, bf_ref))
    fut_ref[:, 128:192] = d(xf, 6, wf_ref, bf_ref)[:, 0:64]


def kernel(inputs, E0, E1, W2, b2, W3, b3, W4, b4, W5, b5, W6, b6, W7, b7):
    f32 = jnp.float32
    idx0 = inputs[:, 0, 0].astype(jnp.int32).reshape(NW, 1, S_PER_W)
    idxh = inputs[:, :HIST, 1].astype(jnp.int32).reshape(NW, NCH_H, CH_H)
    idxf = inputs[:, HIST:, 1].astype(jnp.int32).reshape(NW, NCH_F, CH_F)
    z_tab = jnp.zeros_like(E1)
    e1hi = jnp.concatenate([z_tab, E1], axis=1)
    e1lo = jnp.concatenate([E1, z_tab], axis=1)

    mesh = plsc.VectorSubcoreMesh(core_axis_name="c", subcore_axis_name="s")
    sc_e1 = pl.kernel(
        _sc_gather_e1,
        mesh=mesh,
        out_type=[
            jax.ShapeDtypeStruct((B * HIST, 2 * D), f32),
            jax.ShapeDtypeStruct((B * FUT, 2 * D), f32),
        ],
        scratch_types=[
            pltpu.VMEM((NCH_H, CH_H), jnp.int32),
            pltpu.VMEM((NCH_F, CH_F), jnp.int32),
            pltpu.VMEM((CH_H, 2 * D), f32),
            pltpu.VMEM((CH_H, 2 * D), f32),
            pltpu.SemaphoreType.DMA,
            pltpu.SemaphoreType.DMA,
        ],
    )
    gh, gf = sc_e1(e1hi, e1lo, idxh, idxf)

    sc_e0 = pl.kernel(
        _sc_gather_e0,
        mesh=mesh,
        out_type=[jax.ShapeDtypeStruct((B, D), f32)],
        scratch_types=[
            pltpu.VMEM((1, S_PER_W), jnp.int32),
            pltpu.VMEM((S_PER_W, D), f32),
            pltpu.SemaphoreType.DMA,
        ],
        compiler_params=pltpu.CompilerParams(use_tc_tiling_on_sc=False),
    )
    (s_rows,) = sc_e0(E0, idx0)

    xh_arr = inputs[:, :HIST, :].reshape(B * HIST, NUM_INPUTS)
    xf_arr = inputs[:, HIST:, :].reshape(B * FUT, NUM_INPUTS)
    z64 = jnp.zeros((D,), f32)
    z128 = jnp.zeros((2 * D,), f32)

    def lo(v):
        return jnp.concatenate([v.reshape(D), z64])

    def hi(v):
        return jnp.concatenate([z64, v.reshape(D)])

    wh = jnp.stack([z128, z128, lo(W2), hi(W3), lo(W4), lo(W5), hi(W6), lo(W7)])
    bh = jnp.stack([z128, z128, lo(b2), hi(b3), lo(b4), lo(b5), hi(b6), lo(b7)])
    wf = jnp.stack([z128, z128, z128, z128, z128, hi(W5), lo(W6), z128])
    bf = jnp.stack([z128, z128, z128, z128, z128, hi(b5), lo(b6), z128])

    bb = 8
    bh_rows = bb * HIST
    bf_rows = bb * FUT
    hist_flat, fut_flat = pl.pallas_call(
        _tc_body,
        grid=(B // bb,),
        in_specs=[
            pl.BlockSpec((bh_rows, NUM_INPUTS), lambda b: (b, 0)),
            pl.BlockSpec((bf_rows, NUM_INPUTS), lambda b: (b, 0)),
            pl.BlockSpec((bh_rows, 2 * D), lambda b: (b, 0)),
            pl.BlockSpec((bf_rows, 2 * D), lambda b: (b, 0)),
            pl.BlockSpec((NUM_INPUTS, 2 * D), lambda b: (0, 0)),
            pl.BlockSpec((NUM_INPUTS, 2 * D), lambda b: (0, 0)),
            pl.BlockSpec((NUM_INPUTS, 2 * D), lambda b: (0, 0)),
            pl.BlockSpec((NUM_INPUTS, 2 * D), lambda b: (0, 0)),
        ],
        out_specs=[
            pl.BlockSpec((bh_rows, 7 * D), lambda b: (b, 0)),
            pl.BlockSpec((bf_rows, 3 * D), lambda b: (b, 0)),
        ],
        out_shape=[
            jax.ShapeDtypeStruct((B * HIST, 7 * D), f32),
            jax.ShapeDtypeStruct((B * FUT, 3 * D), f32),
        ],
    )(xh_arr, xf_arr, gh, gf, wh, bh, wf, bf)

    static = s_rows.reshape(B, 1, D)
    historical = hist_flat.reshape(B, HIST, 7, D)
    future = fut_flat.reshape(B, FUT, 3, D)
    return (static, historical, future)


# speedup vs baseline: 3.3269x; 3.3269x over previous
import jax
import jax.numpy as jnp
from jax import lax
from jax.experimental import pallas as pl
from jax.experimental.pallas import tpu as pltpu
from jax.experimental.pallas import tpu_sc as plsc

B = 1024
W = 200
NUM_INPUTS = 8
D = 64
HIST = 150
FUT = W - HIST

NC = 2
NS = 16
NW = NC * NS

CH_H, NCH_H = 120, 40
CH_F, NCH_F = 80, 20
S_PER_W = B // NW

TS = 2


def _sc_gather_e1(e1_hbm, idxh_hbm, idxf_hbm, gh_hbm, gf_hbm,
                  idxh_v, idxf_v, r_a, r_b, sem_a, sem_b):
    wid = lax.axis_index("s") * NC + lax.axis_index("c")

    pltpu.sync_copy(idxh_hbm.at[wid], idxh_v)
    base_h = wid * (NCH_H * CH_H)

    def body_h(k, carry):
        j0 = 2 * k
        j1 = j0 + 1
        c0 = pltpu.async_copy(e1_hbm.at[idxh_v.at[j0]], r_a, sem_a)
        c1 = pltpu.async_copy(e1_hbm.at[idxh_v.at[j1]], r_b, sem_b)
        c0.wait()
        pltpu.sync_copy(r_a, gh_hbm.at[pl.ds(base_h + j0 * CH_H, CH_H)])
        c1.wait()
        pltpu.sync_copy(r_b, gh_hbm.at[pl.ds(base_h + j1 * CH_H, CH_H)])
        return carry

    lax.fori_loop(0, NCH_H // 2, body_h, 0)

    pltpu.sync_copy(idxf_hbm.at[wid], idxf_v)
    base_f = wid * (NCH_F * CH_F)

    def body_f(k, carry):
        j0 = 2 * k
        j1 = j0 + 1
        ra = r_a.at[pl.ds(0, CH_F)]
        rb = r_b.at[pl.ds(0, CH_F)]
        c0 = pltpu.async_copy(e1_hbm.at[idxf_v.at[j0]], ra, sem_a)
        c1 = pltpu.async_copy(e1_hbm.at[idxf_v.at[j1]], rb, sem_b)
        c0.wait()
        pltpu.sync_copy(ra, gf_hbm.at[pl.ds(base_f + j0 * CH_F, CH_F)])
        c1.wait()
        pltpu.sync_copy(rb, gf_hbm.at[pl.ds(base_f + j1 * CH_F, CH_F)])
        return carry

    lax.fori_loop(0, NCH_F // 2, body_f, 0)


def _sc_gather_e0(e0p_hbm, idx0_hbm, s2_hbm, idx0_v, r0_v, sem0):
    wid = lax.axis_index("s") * NC + lax.axis_index("c")
    pltpu.sync_copy(idx0_hbm.at[wid], idx0_v)
    pltpu.async_copy(e0p_hbm.at[idx0_v.at[0]], r0_v, sem0).wait()
    pltpu.sync_copy(r0_v, s2_hbm.at[pl.ds(wid * S_PER_W, S_PER_W)])


HIST_SLOTS = {0: 7, 2: 5, 3: 6, 4: 2, 5: 3, 6: 4}
FUT_SLOTS = {1: 5, 2: 6}


def _tc_hist(xt_ref, gh_ref, wc_ref, bc_ref, hist_ref):
    for tt in range(TS):
        for v in range(7):
            if v == 1:
                slab = gh_ref[pl.ds(tt * B, B), :]
                hist_ref[tt, 1] = jnp.transpose(slab, (1, 0))
            else:
                i = HIST_SLOTS[v]
                xr = xt_ref[tt, i:i + 1, :]
                hist_ref[tt, v] = wc_ref[:, i:i + 1] * xr + bc_ref[:, i:i + 1]


def _tc_fut(xt_ref, gf_ref, wc_ref, bc_ref, fut_ref):
    for tt in range(TS):
        for v in range(3):
            if v == 0:
                slab = gf_ref[pl.ds(tt * B, B), :]
                fut_ref[tt, 0] = jnp.transpose(slab, (1, 0))
            else:
                i = FUT_SLOTS[v]
                xr = xt_ref[tt, i:i + 1, :]
                fut_ref[tt, v] = wc_ref[:, i:i + 1] * xr + bc_ref[:, i:i + 1]


def _tc_static(s2_ref, par_ref, stat_ref):
    s2 = s2_ref[...]
    par = par_ref[...] > 0
    sel = jnp.where(par, s2[:, D:2 * D], s2[:, 0:D])
    stat_ref[0] = jnp.transpose(sel, (1, 0))


def kernel(inputs, E0, E1, W2, b2, W3, b3, W4, b4, W5, b5, W6, b6, W7, b7):
    f32 = jnp.float32
    i32 = jnp.int32
    idx0 = inputs[:, 0, 0].astype(i32)
    idx0q = (idx0 // 2).reshape(NW, 1, S_PER_W)
    par0 = (idx0 % 2).astype(i32).reshape(B, 1)
    idxh = inputs[:, :HIST, 1].astype(i32).T.reshape(NW, NCH_H, CH_H)
    idxf = inputs[:, HIST:, 1].astype(i32).T.reshape(NW, NCH_F, CH_F)

    mesh = plsc.VectorSubcoreMesh(core_axis_name="c", subcore_axis_name="s")
    sc_e1 = pl.kernel(
        _sc_gather_e1,
        mesh=mesh,
        out_type=[
            jax.ShapeDtypeStruct((HIST * B, D), f32),
            jax.ShapeDtypeStruct((FUT * B, D), f32),
        ],
        scratch_types=[
            pltpu.VMEM((NCH_H, CH_H), i32),
            pltpu.VMEM((NCH_F, CH_F), i32),
            pltpu.VMEM((CH_H, D), f32),
            pltpu.VMEM((CH_H, D), f32),
            pltpu.SemaphoreType.DMA,
            pltpu.SemaphoreType.DMA,
        ],
        compiler_params=pltpu.CompilerParams(use_tc_tiling_on_sc=False),
    )
    gh, gf = sc_e1(E1, idxh, idxf)

    sc_e0 = pl.kernel(
        _sc_gather_e0,
        mesh=mesh,
        out_type=[jax.ShapeDtypeStruct((B, 2 * D), f32)],
        scratch_types=[
            pltpu.VMEM((1, S_PER_W), i32),
            pltpu.VMEM((S_PER_W, 2 * D), f32),
            pltpu.SemaphoreType.DMA,
        ],
    )
    (s2_rows,) = sc_e0(E0.reshape(E0.shape[0] // 2, 2 * D), idx0q)

    x_t = jnp.transpose(inputs, (1, 2, 0))
    z64 = jnp.zeros((D,), f32)
    wc = jnp.stack([z64, z64, W2.reshape(D), W3.reshape(D), W4.reshape(D),
                    W5.reshape(D), W6.reshape(D), W7.reshape(D)], axis=1)
    bc = jnp.stack([z64, z64, b2, b3, b4, b5, b6, b7], axis=1)

    hist_phys = pl.pallas_call(
        _tc_hist,
        grid=(HIST // TS,),
        in_specs=[
            pl.BlockSpec((TS, NUM_INPUTS, B), lambda t: (t, 0, 0)),
            pl.BlockSpec((TS * B, D), lambda t: (t, 0)),
            pl.BlockSpec((D, NUM_INPUTS), lambda t: (0, 0)),
            pl.BlockSpec((D, NUM_INPUTS), lambda t: (0, 0)),
        ],
        out_specs=pl.BlockSpec((TS, 7, D, B), lambda t: (t, 0, 0, 0)),
        out_shape=jax.ShapeDtypeStruct((HIST, 7, D, B), f32),
    )(x_t, gh, wc, bc)

    fut_phys = pl.pallas_call(
        _tc_fut,
        grid=(FUT // TS,),
        in_specs=[
            pl.BlockSpec((TS, NUM_INPUTS, B), lambda t: (t + HIST // TS, 0, 0)),
            pl.BlockSpec((TS * B, D), lambda t: (t, 0)),
            pl.BlockSpec((D, NUM_INPUTS), lambda t: (0, 0)),
            pl.BlockSpec((D, NUM_INPUTS), lambda t: (0, 0)),
        ],
        out_specs=pl.BlockSpec((TS, 3, D, B), lambda t: (t, 0, 0, 0)),
        out_shape=jax.ShapeDtypeStruct((FUT, 3, D, B), f32),
    )(x_t, gf, wc, bc)

    stat_phys = pl.pallas_call(
        _tc_static,
        grid=(1,),
        in_specs=[
            pl.BlockSpec((B, 2 * D), lambda t: (0, 0)),
            pl.BlockSpec((B, 1), lambda t: (0, 0)),
        ],
        out_specs=pl.BlockSpec((1, D, B), lambda t: (0, 0, 0)),
        out_shape=jax.ShapeDtypeStruct((1, D, B), f32),
    )(s2_rows, par0)

    static = jnp.transpose(stat_phys, (2, 0, 1))
    historical = jnp.transpose(hist_phys, (3, 0, 1, 2))
    future = jnp.transpose(fut_phys, (3, 0, 1, 2))
    return (static, historical, future)


# speedup vs baseline: 3.3738x; 1.0141x over previous
import jax
import jax.numpy as jnp
from jax import lax
from jax.experimental import pallas as pl
from jax.experimental.pallas import tpu as pltpu
from jax.experimental.pallas import tpu_sc as plsc

B = 1024
W = 200
NUM_INPUTS = 8
D = 64
HIST = 150
FUT = W - HIST

NC = 2
NS = 16
NW = NC * NS

NBUF = 4
CH_H, NCH_H = 120, 40
CH_F, NCH_F = 80, 20
S_PER_W = B // NW

TS = 2


def _gather_rows(e1_hbm, idx_v, out_hbm, bufs, gsems, ssems, base, nch, ch):
    def body(k, carry):
        waits = []
        for q in range(NBUF):
            j = NBUF * k + q
            r = bufs[q].at[pl.ds(0, ch)]
            g = pltpu.async_copy(e1_hbm.at[idx_v.at[j]], r, gsems[q])
            waits.append((g, r, j))
        scs = []
        for g, r, j in waits:
            g.wait()
            scs.append(pltpu.async_copy(
                r, out_hbm.at[pl.ds(base + j * ch, ch)], ssems[q2 := len(scs)]))
        for s in scs:
            s.wait()
        return carry

    lax.fori_loop(0, nch // NBUF, body, 0)


def _sc_gather_gh(e1_hbm, idxh_hbm, gh_hbm, idxh_v,
                  r0, r1, r2, r3, g0, g1, g2, g3, s0, s1, s2, s3):
    wid = lax.axis_index("s") * NC + lax.axis_index("c")
    pltpu.sync_copy(idxh_hbm.at[wid], idxh_v)
    _gather_rows(e1_hbm, idxh_v, gh_hbm, (r0, r1, r2, r3),
                 (g0, g1, g2, g3), (s0, s1, s2, s3),
                 wid * (NCH_H * CH_H), NCH_H, CH_H)


def _sc_gather_gf(e1_hbm, idxf_hbm, gf_hbm, idxf_v,
                  r0, r1, r2, r3, g0, g1, g2, g3, s0, s1, s2, s3):
    wid = lax.axis_index("s") * NC + lax.axis_index("c")
    pltpu.sync_copy(idxf_hbm.at[wid], idxf_v)
    _gather_rows(e1_hbm, idxf_v, gf_hbm, (r0, r1, r2, r3),
                 (g0, g1, g2, g3), (s0, s1, s2, s3),
                 wid * (NCH_F * CH_F), NCH_F, CH_F)


def _sc_gather_e0(e0p_hbm, idx0_hbm, s2_hbm, idx0_v, r0_v, sem0):
    wid = lax.axis_index("s") * NC + lax.axis_index("c")
    pltpu.sync_copy(idx0_hbm.at[wid], idx0_v)
    pltpu.async_copy(e0p_hbm.at[idx0_v.at[0]], r0_v, sem0).wait()
    pltpu.sync_copy(r0_v, s2_hbm.at[pl.ds(wid * S_PER_W, S_PER_W)])


HIST_SLOTS = {0: 7, 2: 5, 3: 6, 4: 2, 5: 3, 6: 4}
FUT_SLOTS = {1: 5, 2: 6}


def _tc_hist(xt_ref, gh_ref, wc_ref, bc_ref, hist_ref):
    for tt in range(TS):
        for v in range(7):
            if v == 1:
                slab = gh_ref[pl.ds(tt * B, B), :]
                hist_ref[tt, 1] = jnp.transpose(slab, (1, 0))
            else:
                i = HIST_SLOTS[v]
                xr = xt_ref[tt, i:i + 1, :]
                hist_ref[tt, v] = wc_ref[:, i:i + 1] * xr + bc_ref[:, i:i + 1]


def _tc_fut(xt_ref, gf_ref, wc_ref, bc_ref, fut_ref):
    for tt in range(TS):
        for v in range(3):
            if v == 0:
                slab = gf_ref[pl.ds(tt * B, B), :]
                fut_ref[tt, 0] = jnp.transpose(slab, (1, 0))
            else:
                i = FUT_SLOTS[v]
                xr = xt_ref[tt, i:i + 1, :]
                fut_ref[tt, v] = wc_ref[:, i:i + 1] * xr + bc_ref[:, i:i + 1]


def _tc_static(s2_ref, par_ref, stat_ref):
    s2 = s2_ref[...]
    par = par_ref[...] > 0
    sel = jnp.where(par, s2[:, D:2 * D], s2[:, 0:D])
    stat_ref[0] = jnp.transpose(sel, (1, 0))


def kernel(inputs, E0, E1, W2, b2, W3, b3, W4, b4, W5, b5, W6, b6, W7, b7):
    f32 = jnp.float32
    i32 = jnp.int32
    idx0 = inputs[:, 0, 0].astype(i32)
    idx0q = (idx0 // 2).reshape(NW, 1, S_PER_W)
    par0 = (idx0 % 2).astype(i32).reshape(B, 1)
    idxh = inputs[:, :HIST, 1].astype(i32).T.reshape(NW, NCH_H, CH_H)
    idxf = inputs[:, HIST:, 1].astype(i32).T.reshape(NW, NCH_F, CH_F)

    mesh = plsc.VectorSubcoreMesh(core_axis_name="c", subcore_axis_name="s")
    e1_scratch = (
        [pltpu.VMEM((NCH_H, CH_H), i32)]
        + [pltpu.VMEM((CH_H, D), f32)] * NBUF
        + [pltpu.SemaphoreType.DMA] * (2 * NBUF)
    )
    sc_gh = pl.kernel(
        _sc_gather_gh,
        mesh=mesh,
        out_type=[jax.ShapeDtypeStruct((HIST * B, D), f32)],
        scratch_types=list(e1_scratch),
        compiler_params=pltpu.CompilerParams(use_tc_tiling_on_sc=False),
    )
    (gh,) = sc_gh(E1, idxh)

    ef_scratch = (
        [pltpu.VMEM((NCH_F, CH_F), i32)]
        + [pltpu.VMEM((CH_H, D), f32)] * NBUF
        + [pltpu.SemaphoreType.DMA] * (2 * NBUF)
    )
    sc_gf = pl.kernel(
        _sc_gather_gf,
        mesh=mesh,
        out_type=[jax.ShapeDtypeStruct((FUT * B, D), f32)],
        scratch_types=list(ef_scratch),
        compiler_params=pltpu.CompilerParams(use_tc_tiling_on_sc=False),
    )
    (gf,) = sc_gf(E1, idxf)

    sc_e0 = pl.kernel(
        _sc_gather_e0,
        mesh=mesh,
        out_type=[jax.ShapeDtypeStruct((B, 2 * D), f32)],
        scratch_types=[
            pltpu.VMEM((1, S_PER_W), i32),
            pltpu.VMEM((S_PER_W, 2 * D), f32),
            pltpu.SemaphoreType.DMA,
        ],
    )
    (s2_rows,) = sc_e0(E0.reshape(E0.shape[0] // 2, 2 * D), idx0q)

    x_t = jnp.transpose(inputs, (1, 2, 0))
    z64 = jnp.zeros((D,), f32)
    wc = jnp.stack([z64, z64, W2.reshape(D), W3.reshape(D), W4.reshape(D),
                    W5.reshape(D), W6.reshape(D), W7.reshape(D)], axis=1)
    bc = jnp.stack([z64, z64, b2, b3, b4, b5, b6, b7], axis=1)

    hist_phys = pl.pallas_call(
        _tc_hist,
        grid=(HIST // TS,),
        in_specs=[
            pl.BlockSpec((TS, NUM_INPUTS, B), lambda t: (t, 0, 0)),
            pl.BlockSpec((TS * B, D), lambda t: (t, 0)),
            pl.BlockSpec((D, NUM_INPUTS), lambda t: (0, 0)),
            pl.BlockSpec((D, NUM_INPUTS), lambda t: (0, 0)),
        ],
        out_specs=pl.BlockSpec((TS, 7, D, B), lambda t: (t, 0, 0, 0)),
        out_shape=jax.ShapeDtypeStruct((HIST, 7, D, B), f32),
    )(x_t, gh, wc, bc)

    fut_phys = pl.pallas_call(
        _tc_fut,
        grid=(FUT // TS,),
        in_specs=[
            pl.BlockSpec((TS, NUM_INPUTS, B), lambda t: (t + HIST // TS, 0, 0)),
            pl.BlockSpec((TS * B, D), lambda t: (t, 0)),
            pl.BlockSpec((D, NUM_INPUTS), lambda t: (0, 0)),
            pl.BlockSpec((D, NUM_INPUTS), lambda t: (0, 0)),
        ],
        out_specs=pl.BlockSpec((TS, 3, D, B), lambda t: (t, 0, 0, 0)),
        out_shape=jax.ShapeDtypeStruct((FUT, 3, D, B), f32),
    )(x_t, gf, wc, bc)

    stat_phys = pl.pallas_call(
        _tc_static,
        grid=(1,),
        in_specs=[
            pl.BlockSpec((B, 2 * D), lambda t: (0, 0)),
            pl.BlockSpec((B, 1), lambda t: (0, 0)),
        ],
        out_specs=pl.BlockSpec((1, D, B), lambda t: (0, 0, 0)),
        out_shape=jax.ShapeDtypeStruct((1, D, B), f32),
    )(s2_rows, par0)

    static = jnp.transpose(stat_phys, (2, 0, 1))
    historical = jnp.transpose(hist_phys, (3, 0, 1, 2))
    future = jnp.transpose(fut_phys, (3, 0, 1, 2))
    return (static, historical, future)


# speedup vs baseline: 3.4136x; 1.0118x over previous
import jax
import jax.numpy as jnp
from jax import lax
from jax.experimental import pallas as pl
from jax.experimental.pallas import tpu as pltpu
from jax.experimental.pallas import tpu_sc as plsc

B = 1024
W = 200
NUM_INPUTS = 8
D = 64
HIST = 150
FUT = W - HIST

NC = 2
NS = 16
NW = NC * NS

CH_H, NCH_H = 120, 40
CH_F, NCH_F = 80, 20
S_PER_W = B // NW

TS = 2


def _gather_rows(e1_hbm, idx_v, out_hbm, r_a, r_b, sem_a, sem_b, base, nch, ch):
    def body(k, carry):
        j0 = 2 * k
        j1 = j0 + 1
        ra = r_a.at[pl.ds(0, ch)]
        rb = r_b.at[pl.ds(0, ch)]
        c0 = pltpu.async_copy(e1_hbm.at[idx_v.at[j0]], ra, sem_a)
        c1 = pltpu.async_copy(e1_hbm.at[idx_v.at[j1]], rb, sem_b)
        c0.wait()
        pltpu.sync_copy(ra, out_hbm.at[pl.ds(base + j0 * ch, ch)])
        c1.wait()
        pltpu.sync_copy(rb, out_hbm.at[pl.ds(base + j1 * ch, ch)])
        return carry

    lax.fori_loop(0, nch // 2, body, 0)


def _sc_gather_gh(e1_hbm, idxh_hbm, gh_hbm, idxh_v, r_a, r_b, sem_a, sem_b):
    wid = lax.axis_index("s") * NC + lax.axis_index("c")
    pltpu.sync_copy(idxh_hbm.at[wid], idxh_v)
    _gather_rows(e1_hbm, idxh_v, gh_hbm, r_a, r_b, sem_a, sem_b,
                 wid * (NCH_H * CH_H), NCH_H, CH_H)


def _sc_gather_gf(e1_hbm, idxf_hbm, gf_hbm, idxf_v, r_a, r_b, sem_a, sem_b):
    wid = lax.axis_index("s") * NC + lax.axis_index("c")
    pltpu.sync_copy(idxf_hbm.at[wid], idxf_v)
    _gather_rows(e1_hbm, idxf_v, gf_hbm, r_a, r_b, sem_a, sem_b,
                 wid * (NCH_F * CH_F), NCH_F, CH_F)


def _sc_gather_e0(e0p_hbm, idx0_hbm, s2_hbm, idx0_v, r0_v, sem0):
    wid = lax.axis_index("s") * NC + lax.axis_index("c")
    pltpu.sync_copy(idx0_hbm.at[wid], idx0_v)
    pltpu.async_copy(e0p_hbm.at[idx0_v.at[0]], r0_v, sem0).wait()
    pltpu.sync_copy(r0_v, s2_hbm.at[pl.ds(wid * S_PER_W, S_PER_W)])


HIST_SLOTS = {0: 7, 2: 5, 3: 6, 4: 2, 5: 3, 6: 4}
FUT_SLOTS = {1: 5, 2: 6}


def _tc_hist(xt_ref, gh_ref, wc_ref, bc_ref, hist_ref):
    for tt in range(TS):
        for v in range(7):
            if v == 1:
                slab = gh_ref[pl.ds(tt * B, B), :]
                hist_ref[tt, 1] = jnp.transpose(slab, (1, 0))
            else:
                i = HIST_SLOTS[v]
                xr = xt_ref[tt, i:i + 1, :]
                hist_ref[tt, v] = wc_ref[:, i:i + 1] * xr + bc_ref[:, i:i + 1]


def _tc_fut(xt_ref, gf_ref, wc_ref, bc_ref, fut_ref):
    for tt in range(TS):
        for v in range(3):
            if v == 0:
                slab = gf_ref[pl.ds(tt * B, B), :]
                fut_ref[tt, 0] = jnp.transpose(slab, (1, 0))
            else:
                i = FUT_SLOTS[v]
                xr = xt_ref[tt, i:i + 1, :]
                fut_ref[tt, v] = wc_ref[:, i:i + 1] * xr + bc_ref[:, i:i + 1]


def _tc_static(s2_ref, par_ref, stat_ref):
    s2 = s2_ref[...]
    par = par_ref[...] > 0
    sel = jnp.where(par, s2[:, D:2 * D], s2[:, 0:D])
    stat_ref[0] = jnp.transpose(sel, (1, 0))


def kernel(inputs, E0, E1, W2, b2, W3, b3, W4, b4, W5, b5, W6, b6, W7, b7):
    f32 = jnp.float32
    i32 = jnp.int32
    idx0 = inputs[:, 0, 0].astype(i32)
    idx0q = (idx0 // 2).reshape(NW, 1, S_PER_W)
    par0 = (idx0 % 2).astype(i32).reshape(B, 1)
    idxh = inputs[:, :HIST, 1].astype(i32).T.reshape(NW, NCH_H, CH_H)
    idxf = inputs[:, HIST:, 1].astype(i32).T.reshape(NW, NCH_F, CH_F)

    mesh = plsc.VectorSubcoreMesh(core_axis_name="c", subcore_axis_name="s")
    e1_scratch = (
        [pltpu.VMEM((NCH_H, CH_H), i32)]
        + [pltpu.VMEM((CH_H, D), f32)] * 2
        + [pltpu.SemaphoreType.DMA] * 2
    )
    sc_gh = pl.kernel(
        _sc_gather_gh,
        mesh=mesh,
        out_type=[jax.ShapeDtypeStruct((HIST * B, D), f32)],
        scratch_types=list(e1_scratch),
        compiler_params=pltpu.CompilerParams(use_tc_tiling_on_sc=False),
    )
    (gh,) = sc_gh(E1, idxh)

    ef_scratch = (
        [pltpu.VMEM((NCH_F, CH_F), i32)]
        + [pltpu.VMEM((CH_H, D), f32)] * 2
        + [pltpu.SemaphoreType.DMA] * 2
    )
    sc_gf = pl.kernel(
        _sc_gather_gf,
        mesh=mesh,
        out_type=[jax.ShapeDtypeStruct((FUT * B, D), f32)],
        scratch_types=list(ef_scratch),
        compiler_params=pltpu.CompilerParams(use_tc_tiling_on_sc=False),
    )
    (gf,) = sc_gf(E1, idxf)

    sc_e0 = pl.kernel(
        _sc_gather_e0,
        mesh=mesh,
        out_type=[jax.ShapeDtypeStruct((B, 2 * D), f32)],
        scratch_types=[
            pltpu.VMEM((1, S_PER_W), i32),
            pltpu.VMEM((S_PER_W, 2 * D), f32),
            pltpu.SemaphoreType.DMA,
        ],
    )
    (s2_rows,) = sc_e0(E0.reshape(E0.shape[0] // 2, 2 * D), idx0q)

    x_t = jnp.transpose(inputs, (1, 2, 0))
    z64 = jnp.zeros((D,), f32)
    wc = jnp.stack([z64, z64, W2.reshape(D), W3.reshape(D), W4.reshape(D),
                    W5.reshape(D), W6.reshape(D), W7.reshape(D)], axis=1)
    bc = jnp.stack([z64, z64, b2, b3, b4, b5, b6, b7], axis=1)

    hist_phys = pl.pallas_call(
        _tc_hist,
        grid=(HIST // TS,),
        in_specs=[
            pl.BlockSpec((TS, NUM_INPUTS, B), lambda t: (t, 0, 0)),
            pl.BlockSpec((TS * B, D), lambda t: (t, 0)),
            pl.BlockSpec((D, NUM_INPUTS), lambda t: (0, 0)),
            pl.BlockSpec((D, NUM_INPUTS), lambda t: (0, 0)),
        ],
        out_specs=pl.BlockSpec((TS, 7, D, B), lambda t: (t, 0, 0, 0)),
        out_shape=jax.ShapeDtypeStruct((HIST, 7, D, B), f32),
    )(x_t, gh, wc, bc)

    fut_phys = pl.pallas_call(
        _tc_fut,
        grid=(FUT // TS,),
        in_specs=[
            pl.BlockSpec((TS, NUM_INPUTS, B), lambda t: (t + HIST // TS, 0, 0)),
            pl.BlockSpec((TS * B, D), lambda t: (t, 0)),
            pl.BlockSpec((D, NUM_INPUTS), lambda t: (0, 0)),
            pl.BlockSpec((D, NUM_INPUTS), lambda t: (0, 0)),
        ],
        out_specs=pl.BlockSpec((TS, 3, D, B), lambda t: (t, 0, 0, 0)),
        out_shape=jax.ShapeDtypeStruct((FUT, 3, D, B), f32),
    )(x_t, gf, wc, bc)

    stat_phys = pl.pallas_call(
        _tc_static,
        grid=(1,),
        in_specs=[
            pl.BlockSpec((B, 2 * D), lambda t: (0, 0)),
            pl.BlockSpec((B, 1), lambda t: (0, 0)),
        ],
        out_specs=pl.BlockSpec((1, D, B), lambda t: (0, 0, 0)),
        out_shape=jax.ShapeDtypeStruct((1, D, B), f32),
    )(s2_rows, par0)

    static = jnp.transpose(stat_phys, (2, 0, 1))
    historical = jnp.transpose(hist_phys, (3, 0, 1, 2))
    future = jnp.transpose(fut_phys, (3, 0, 1, 2))
    return (static, historical, future)


# speedup vs baseline: 3.6573x; 1.0714x over previous
import jax
import jax.numpy as jnp
from jax import lax
from jax.experimental import pallas as pl
from jax.experimental.pallas import tpu as pltpu
from jax.experimental.pallas import tpu_sc as plsc

B = 1024
W = 200
NUM_INPUTS = 8
D = 64
HIST = 150
FUT = W - HIST

NC = 2
NS = 16
NW = NC * NS

CH_H, NCH_H = 120, 40
CH_F, NCH_F = 80, 20
S_PER_W = B // NW

TS = 5


def _gather_rows(e1_hbm, idx_v, out_hbm, r_a, r_b, sem_a, sem_b, base, nch, ch):
    def body(k, carry):
        j0 = 2 * k
        j1 = j0 + 1
        ra = r_a.at[pl.ds(0, ch)]
        rb = r_b.at[pl.ds(0, ch)]
        c0 = pltpu.async_copy(e1_hbm.at[idx_v.at[j0]], ra, sem_a)
        c1 = pltpu.async_copy(e1_hbm.at[idx_v.at[j1]], rb, sem_b)
        c0.wait()
        pltpu.sync_copy(ra, out_hbm.at[pl.ds(base + j0 * ch, ch)])
        c1.wait()
        pltpu.sync_copy(rb, out_hbm.at[pl.ds(base + j1 * ch, ch)])
        return carry

    lax.fori_loop(0, nch // 2, body, 0)


def _sc_gather_gh(e1_hbm, idxh_hbm, gh_hbm, idxh_v, r_a, r_b, sem_a, sem_b):
    wid = lax.axis_index("s") * NC + lax.axis_index("c")
    pltpu.sync_copy(idxh_hbm.at[wid], idxh_v)
    _gather_rows(e1_hbm, idxh_v, gh_hbm, r_a, r_b, sem_a, sem_b,
                 wid * (NCH_H * CH_H), NCH_H, CH_H)


def _sc_gather_gf(e1_hbm, idxf_hbm, gf_hbm, idxf_v, r_a, r_b, sem_a, sem_b):
    wid = lax.axis_index("s") * NC + lax.axis_index("c")
    pltpu.sync_copy(idxf_hbm.at[wid], idxf_v)
    _gather_rows(e1_hbm, idxf_v, gf_hbm, r_a, r_b, sem_a, sem_b,
                 wid * (NCH_F * CH_F), NCH_F, CH_F)


def _sc_gather_e0(e0p_hbm, idx0_hbm, s2_hbm, idx0_v, r0_v, sem0):
    wid = lax.axis_index("s") * NC + lax.axis_index("c")
    pltpu.sync_copy(idx0_hbm.at[wid], idx0_v)
    pltpu.async_copy(e0p_hbm.at[idx0_v.at[0]], r0_v, sem0).wait()
    pltpu.sync_copy(r0_v, s2_hbm.at[pl.ds(wid * S_PER_W, S_PER_W)])


HIST_SLOTS = {0: 7, 2: 5, 3: 6, 4: 2, 5: 3, 6: 4}
FUT_SLOTS = {1: 5, 2: 6}


def _tc_hist(xt_ref, gh_ref, wc_ref, bc_ref, hist_ref):
    for tt in range(TS):
        for v in range(7):
            if v == 1:
                slab = gh_ref[pl.ds(tt * B, B), :]
                hist_ref[tt, 1] = jnp.transpose(slab, (1, 0))
            else:
                i = HIST_SLOTS[v]
                xr = xt_ref[tt, i:i + 1, :]
                hist_ref[tt, v] = wc_ref[:, i:i + 1] * xr + bc_ref[:, i:i + 1]


def _tc_fut(xt_ref, gf_ref, wc_ref, bc_ref, fut_ref):
    for tt in range(TS):
        for v in range(3):
            if v == 0:
                slab = gf_ref[pl.ds(tt * B, B), :]
                fut_ref[tt, 0] = jnp.transpose(slab, (1, 0))
            else:
                i = FUT_SLOTS[v]
                xr = xt_ref[tt, i:i + 1, :]
                fut_ref[tt, v] = wc_ref[:, i:i + 1] * xr + bc_ref[:, i:i + 1]


def _tc_static(s2_ref, par_ref, stat_ref):
    s2 = s2_ref[...]
    par = par_ref[...] > 0
    sel = jnp.where(par, s2[:, D:2 * D], s2[:, 0:D])
    stat_ref[0] = jnp.transpose(sel, (1, 0))


def kernel(inputs, E0, E1, W2, b2, W3, b3, W4, b4, W5, b5, W6, b6, W7, b7):
    f32 = jnp.float32
    i32 = jnp.int32
    idx0 = inputs[:, 0, 0].astype(i32)
    idx0q = (idx0 // 2).reshape(NW, 1, S_PER_W)
    par0 = (idx0 % 2).astype(i32).reshape(B, 1)
    idxh = inputs[:, :HIST, 1].astype(i32).T.reshape(NW, NCH_H, CH_H)
    idxf = inputs[:, HIST:, 1].astype(i32).T.reshape(NW, NCH_F, CH_F)

    mesh = plsc.VectorSubcoreMesh(core_axis_name="c", subcore_axis_name="s")
    e1_scratch = (
        [pltpu.VMEM((NCH_H, CH_H), i32)]
        + [pltpu.VMEM((CH_H, D), f32)] * 2
        + [pltpu.SemaphoreType.DMA] * 2
    )
    sc_gh = pl.kernel(
        _sc_gather_gh,
        mesh=mesh,
        out_type=[jax.ShapeDtypeStruct((HIST * B, D), f32)],
        scratch_types=list(e1_scratch),
        compiler_params=pltpu.CompilerParams(use_tc_tiling_on_sc=False),
    )
    (gh,) = sc_gh(E1, idxh)

    ef_scratch = (
        [pltpu.VMEM((NCH_F, CH_F), i32)]
        + [pltpu.VMEM((CH_H, D), f32)] * 2
        + [pltpu.SemaphoreType.DMA] * 2
    )
    sc_gf = pl.kernel(
        _sc_gather_gf,
        mesh=mesh,
        out_type=[jax.ShapeDtypeStruct((FUT * B, D), f32)],
        scratch_types=list(ef_scratch),
        compiler_params=pltpu.CompilerParams(use_tc_tiling_on_sc=False),
    )
    (gf,) = sc_gf(E1, idxf)

    sc_e0 = pl.kernel(
        _sc_gather_e0,
        mesh=mesh,
        out_type=[jax.ShapeDtypeStruct((B, 2 * D), f32)],
        scratch_types=[
            pltpu.VMEM((1, S_PER_W), i32),
            pltpu.VMEM((S_PER_W, 2 * D), f32),
            pltpu.SemaphoreType.DMA,
        ],
    )
    (s2_rows,) = sc_e0(E0.reshape(E0.shape[0] // 2, 2 * D), idx0q)

    x_t = jnp.transpose(inputs, (1, 2, 0))
    z64 = jnp.zeros((D,), f32)
    wc = jnp.stack([z64, z64, W2.reshape(D), W3.reshape(D), W4.reshape(D),
                    W5.reshape(D), W6.reshape(D), W7.reshape(D)], axis=1)
    bc = jnp.stack([z64, z64, b2, b3, b4, b5, b6, b7], axis=1)

    hist_phys = pl.pallas_call(
        _tc_hist,
        grid=(HIST // TS,),
        in_specs=[
            pl.BlockSpec((TS, NUM_INPUTS, B), lambda t: (t, 0, 0)),
            pl.BlockSpec((TS * B, D), lambda t: (t, 0)),
            pl.BlockSpec((D, NUM_INPUTS), lambda t: (0, 0)),
            pl.BlockSpec((D, NUM_INPUTS), lambda t: (0, 0)),
        ],
        out_specs=pl.BlockSpec((TS, 7, D, B), lambda t: (t, 0, 0, 0)),
        out_shape=jax.ShapeDtypeStruct((HIST, 7, D, B), f32),
    )(x_t, gh, wc, bc)

    fut_phys = pl.pallas_call(
        _tc_fut,
        grid=(FUT // TS,),
        in_specs=[
            pl.BlockSpec((TS, NUM_INPUTS, B), lambda t: (t + HIST // TS, 0, 0)),
            pl.BlockSpec((TS * B, D), lambda t: (t, 0)),
            pl.BlockSpec((D, NUM_INPUTS), lambda t: (0, 0)),
            pl.BlockSpec((D, NUM_INPUTS), lambda t: (0, 0)),
        ],
        out_specs=pl.BlockSpec((TS, 3, D, B), lambda t: (t, 0, 0, 0)),
        out_shape=jax.ShapeDtypeStruct((FUT, 3, D, B), f32),
    )(x_t, gf, wc, bc)

    stat_phys = pl.pallas_call(
        _tc_static,
        grid=(1,),
        in_specs=[
            pl.BlockSpec((B, 2 * D), lambda t: (0, 0)),
            pl.BlockSpec((B, 1), lambda t: (0, 0)),
        ],
        out_specs=pl.BlockSpec((1, D, B), lambda t: (0, 0, 0)),
        out_shape=jax.ShapeDtypeStruct((1, D, B), f32),
    )(s2_rows, par0)

    static = jnp.transpose(stat_phys, (2, 0, 1))
    historical = jnp.transpose(hist_phys, (3, 0, 1, 2))
    future = jnp.transpose(fut_phys, (3, 0, 1, 2))
    return (static, historical, future)


# speedup vs baseline: 3.6890x; 1.0087x over previous
import jax
import jax.numpy as jnp
from jax import lax
from jax.experimental import pallas as pl
from jax.experimental.pallas import tpu as pltpu
from jax.experimental.pallas import tpu_sc as plsc

B = 1024
W = 200
NUM_INPUTS = 8
D = 64
HIST = 150
FUT = W - HIST

NC = 2
NS = 16
NW = NC * NS

CH_H, NCH_H = 120, 40
CH_F, NCH_F = 80, 20
S_PER_W = B // NW

TS = 10


def _gather_rows(e1_hbm, idx_v, out_hbm, r_a, r_b, sem_a, sem_b, base, nch, ch):
    def body(k, carry):
        j0 = 2 * k
        j1 = j0 + 1
        ra = r_a.at[pl.ds(0, ch)]
        rb = r_b.at[pl.ds(0, ch)]
        c0 = pltpu.async_copy(e1_hbm.at[idx_v.at[j0]], ra, sem_a)
        c1 = pltpu.async_copy(e1_hbm.at[idx_v.at[j1]], rb, sem_b)
        c0.wait()
        pltpu.sync_copy(ra, out_hbm.at[pl.ds(base + j0 * ch, ch)])
        c1.wait()
        pltpu.sync_copy(rb, out_hbm.at[pl.ds(base + j1 * ch, ch)])
        return carry

    lax.fori_loop(0, nch // 2, body, 0)


def _sc_gather_gh(e1_hbm, idxh_hbm, gh_hbm, idxh_v, r_a, r_b, sem_a, sem_b):
    wid = lax.axis_index("s") * NC + lax.axis_index("c")
    pltpu.sync_copy(idxh_hbm.at[wid], idxh_v)
    _gather_rows(e1_hbm, idxh_v, gh_hbm, r_a, r_b, sem_a, sem_b,
                 wid * (NCH_H * CH_H), NCH_H, CH_H)


def _sc_gather_gf(e1_hbm, idxf_hbm, gf_hbm, idxf_v, r_a, r_b, sem_a, sem_b):
    wid = lax.axis_index("s") * NC + lax.axis_index("c")
    pltpu.sync_copy(idxf_hbm.at[wid], idxf_v)
    _gather_rows(e1_hbm, idxf_v, gf_hbm, r_a, r_b, sem_a, sem_b,
                 wid * (NCH_F * CH_F), NCH_F, CH_F)


def _sc_gather_e0(e0p_hbm, idx0_hbm, s2_hbm, idx0_v, r0_v, sem0):
    wid = lax.axis_index("s") * NC + lax.axis_index("c")
    pltpu.sync_copy(idx0_hbm.at[wid], idx0_v)
    pltpu.async_copy(e0p_hbm.at[idx0_v.at[0]], r0_v, sem0).wait()
    pltpu.sync_copy(r0_v, s2_hbm.at[pl.ds(wid * S_PER_W, S_PER_W)])


HIST_SLOTS = {0: 7, 2: 5, 3: 6, 4: 2, 5: 3, 6: 4}
FUT_SLOTS = {1: 5, 2: 6}


def _tc_hist(xt_ref, gh_ref, wc_ref, bc_ref, hist_ref):
    for tt in range(TS):
        for v in range(7):
            if v == 1:
                slab = gh_ref[pl.ds(tt * B, B), :]
                hist_ref[tt, 1] = jnp.transpose(slab, (1, 0))
            else:
                i = HIST_SLOTS[v]
                xr = xt_ref[tt, i:i + 1, :]
                hist_ref[tt, v] = wc_ref[:, i:i + 1] * xr + bc_ref[:, i:i + 1]


def _tc_fut(xt_ref, gf_ref, wc_ref, bc_ref, fut_ref):
    for tt in range(TS):
        for v in range(3):
            if v == 0:
                slab = gf_ref[pl.ds(tt * B, B), :]
                fut_ref[tt, 0] = jnp.transpose(slab, (1, 0))
            else:
                i = FUT_SLOTS[v]
                xr = xt_ref[tt, i:i + 1, :]
                fut_ref[tt, v] = wc_ref[:, i:i + 1] * xr + bc_ref[:, i:i + 1]


def _tc_static(s2_ref, par_ref, stat_ref):
    s2 = s2_ref[...]
    par = par_ref[...] > 0
    sel = jnp.where(par, s2[:, D:2 * D], s2[:, 0:D])
    stat_ref[0] = jnp.transpose(sel, (1, 0))


def kernel(inputs, E0, E1, W2, b2, W3, b3, W4, b4, W5, b5, W6, b6, W7, b7):
    f32 = jnp.float32
    i32 = jnp.int32
    idx0 = inputs[:, 0, 0].astype(i32)
    idx0q = (idx0 // 2).reshape(NW, 1, S_PER_W)
    par0 = (idx0 % 2).astype(i32).reshape(B, 1)
    idxh = inputs[:, :HIST, 1].astype(i32).T.reshape(NW, NCH_H, CH_H)
    idxf = inputs[:, HIST:, 1].astype(i32).T.reshape(NW, NCH_F, CH_F)

    mesh = plsc.VectorSubcoreMesh(core_axis_name="c", subcore_axis_name="s")
    e1_scratch = (
        [pltpu.VMEM((NCH_H, CH_H), i32)]
        + [pltpu.VMEM((CH_H, D), f32)] * 2
        + [pltpu.SemaphoreType.DMA] * 2
    )
    sc_gh = pl.kernel(
        _sc_gather_gh,
        mesh=mesh,
        out_type=[jax.ShapeDtypeStruct((HIST * B, D), f32)],
        scratch_types=list(e1_scratch),
        compiler_params=pltpu.CompilerParams(use_tc_tiling_on_sc=False),
    )
    (gh,) = sc_gh(E1, idxh)

    ef_scratch = (
        [pltpu.VMEM((NCH_F, CH_F), i32)]
        + [pltpu.VMEM((CH_H, D), f32)] * 2
        + [pltpu.SemaphoreType.DMA] * 2
    )
    sc_gf = pl.kernel(
        _sc_gather_gf,
        mesh=mesh,
        out_type=[jax.ShapeDtypeStruct((FUT * B, D), f32)],
        scratch_types=list(ef_scratch),
        compiler_params=pltpu.CompilerParams(use_tc_tiling_on_sc=False),
    )
    (gf,) = sc_gf(E1, idxf)

    sc_e0 = pl.kernel(
        _sc_gather_e0,
        mesh=mesh,
        out_type=[jax.ShapeDtypeStruct((B, 2 * D), f32)],
        scratch_types=[
            pltpu.VMEM((1, S_PER_W), i32),
            pltpu.VMEM((S_PER_W, 2 * D), f32),
            pltpu.SemaphoreType.DMA,
        ],
    )
    (s2_rows,) = sc_e0(E0.reshape(E0.shape[0] // 2, 2 * D), idx0q)

    x_t = jnp.transpose(inputs, (1, 2, 0))
    z64 = jnp.zeros((D,), f32)
    wc = jnp.stack([z64, z64, W2.reshape(D), W3.reshape(D), W4.reshape(D),
                    W5.reshape(D), W6.reshape(D), W7.reshape(D)], axis=1)
    bc = jnp.stack([z64, z64, b2, b3, b4, b5, b6, b7], axis=1)

    hist_phys = pl.pallas_call(
        _tc_hist,
        grid=(HIST // TS,),
        in_specs=[
            pl.BlockSpec((TS, NUM_INPUTS, B), lambda t: (t, 0, 0)),
            pl.BlockSpec((TS * B, D), lambda t: (t, 0)),
            pl.BlockSpec((D, NUM_INPUTS), lambda t: (0, 0)),
            pl.BlockSpec((D, NUM_INPUTS), lambda t: (0, 0)),
        ],
        out_specs=pl.BlockSpec((TS, 7, D, B), lambda t: (t, 0, 0, 0)),
        out_shape=jax.ShapeDtypeStruct((HIST, 7, D, B), f32),
    )(x_t, gh, wc, bc)

    fut_phys = pl.pallas_call(
        _tc_fut,
        grid=(FUT // TS,),
        in_specs=[
            pl.BlockSpec((TS, NUM_INPUTS, B), lambda t: (t + HIST // TS, 0, 0)),
            pl.BlockSpec((TS * B, D), lambda t: (t, 0)),
            pl.BlockSpec((D, NUM_INPUTS), lambda t: (0, 0)),
            pl.BlockSpec((D, NUM_INPUTS), lambda t: (0, 0)),
        ],
        out_specs=pl.BlockSpec((TS, 3, D, B), lambda t: (t, 0, 0, 0)),
        out_shape=jax.ShapeDtypeStruct((FUT, 3, D, B), f32),
    )(x_t, gf, wc, bc)

    stat_phys = pl.pallas_call(
        _tc_static,
        grid=(1,),
        in_specs=[
            pl.BlockSpec((B, 2 * D), lambda t: (0, 0)),
            pl.BlockSpec((B, 1), lambda t: (0, 0)),
        ],
        out_specs=pl.BlockSpec((1, D, B), lambda t: (0, 0, 0)),
        out_shape=jax.ShapeDtypeStruct((1, D, B), f32),
    )(s2_rows, par0)

    static = jnp.transpose(stat_phys, (2, 0, 1))
    historical = jnp.transpose(hist_phys, (3, 0, 1, 2))
    future = jnp.transpose(fut_phys, (3, 0, 1, 2))
    return (static, historical, future)


# speedup vs baseline: 3.6974x; 1.0023x over previous
import jax
import jax.numpy as jnp
from jax import lax
from jax.experimental import pallas as pl
from jax.experimental.pallas import tpu as pltpu
from jax.experimental.pallas import tpu_sc as plsc

B = 1024
W = 200
NUM_INPUTS = 8
D = 64
HIST = 150
FUT = W - HIST

NC = 2
NS = 16
NW = NC * NS

CH_H, NCH_H = 120, 40
CH_F, NCH_F = 80, 20
S_PER_W = B // NW

TS = 10


def _gather_rows(e1_hbm, idx_v, out_hbm, r_a, r_b, sem_a, sem_b, base, nch, ch):
    def body(k, carry):
        j0 = 2 * k
        j1 = j0 + 1
        ra = r_a.at[pl.ds(0, ch)]
        rb = r_b.at[pl.ds(0, ch)]
        c0 = pltpu.async_copy(e1_hbm.at[idx_v.at[j0]], ra, sem_a)
        c1 = pltpu.async_copy(e1_hbm.at[idx_v.at[j1]], rb, sem_b)
        c0.wait()
        pltpu.sync_copy(ra, out_hbm.at[pl.ds(base + j0 * ch, ch)])
        c1.wait()
        pltpu.sync_copy(rb, out_hbm.at[pl.ds(base + j1 * ch, ch)])
        return carry

    lax.fori_loop(0, nch // 2, body, 0)


def _sc_gather_gh(e1_hbm, idxh_hbm, gh_hbm, idxh_v, r_a, r_b, sem_a, sem_b):
    wid = lax.axis_index("s") * NC + lax.axis_index("c")
    pltpu.sync_copy(idxh_hbm.at[wid], idxh_v)
    _gather_rows(e1_hbm, idxh_v, gh_hbm, r_a, r_b, sem_a, sem_b,
                 wid * (NCH_H * CH_H), NCH_H, CH_H)


def _sc_gather_gf(e1_hbm, idxf_hbm, gf_hbm, idxf_v, r_a, r_b, sem_a, sem_b):
    wid = lax.axis_index("s") * NC + lax.axis_index("c")
    pltpu.sync_copy(idxf_hbm.at[wid], idxf_v)
    _gather_rows(e1_hbm, idxf_v, gf_hbm, r_a, r_b, sem_a, sem_b,
                 wid * (NCH_F * CH_F), NCH_F, CH_F)


def _sc_gather_e0(e0p_hbm, idx0_hbm, s2_hbm, idx0_v, r0_v, sem0):
    wid = lax.axis_index("s") * NC + lax.axis_index("c")
    pltpu.sync_copy(idx0_hbm.at[wid], idx0_v)
    pltpu.async_copy(e0p_hbm.at[idx0_v.at[0]], r0_v, sem0).wait()
    pltpu.sync_copy(r0_v, s2_hbm.at[pl.ds(wid * S_PER_W, S_PER_W)])


HIST_SLOTS = {0: 7, 2: 5, 3: 6, 4: 2, 5: 3, 6: 4}
FUT_SLOTS = {1: 5, 2: 6}


def _tc_hist(xt_ref, gh_ref, wc_ref, bc_ref, hist_ref):
    for tt in range(TS):
        for v in range(7):
            if v == 1:
                slab = gh_ref[pl.ds(tt * B, B), :]
                hist_ref[tt, 1] = jnp.transpose(slab, (1, 0))
            else:
                i = HIST_SLOTS[v]
                xr = xt_ref[tt, i:i + 1, :]
                hist_ref[tt, v] = wc_ref[:, i:i + 1] * xr + bc_ref[:, i:i + 1]


def _tc_fut(xt_ref, gf_ref, wc_ref, bc_ref, fut_ref):
    for tt in range(TS):
        for v in range(3):
            if v == 0:
                slab = gf_ref[pl.ds(tt * B, B), :]
                fut_ref[tt, 0] = jnp.transpose(slab, (1, 0))
            else:
                i = FUT_SLOTS[v]
                xr = xt_ref[tt, i:i + 1, :]
                fut_ref[tt, v] = wc_ref[:, i:i + 1] * xr + bc_ref[:, i:i + 1]


def _tc_static(s2_ref, par_ref, stat_ref):
    s2 = s2_ref[...]
    par = par_ref[...] > 0
    sel = jnp.where(par, s2[:, D:2 * D], s2[:, 0:D])
    stat_ref[0] = jnp.transpose(sel, (1, 0))


def kernel(inputs, E0, E1, W2, b2, W3, b3, W4, b4, W5, b5, W6, b6, W7, b7):
    f32 = jnp.float32
    i32 = jnp.int32
    idx0 = inputs[:, 0, 0].astype(i32)
    idx0q = (idx0 // 2).reshape(NW, 1, S_PER_W)
    par0 = (idx0 % 2).astype(i32).reshape(B, 1)
    idxh = inputs[:, :HIST, 1].astype(i32).T.reshape(NW, NCH_H, CH_H)
    idxf = inputs[:, HIST:, 1].astype(i32).T.reshape(NW, NCH_F, CH_F)

    mesh = plsc.VectorSubcoreMesh(core_axis_name="c", subcore_axis_name="s")
    ef_scratch = (
        [pltpu.VMEM((NCH_F, CH_F), i32)]
        + [pltpu.VMEM((CH_H, D), f32)] * 2
        + [pltpu.SemaphoreType.DMA] * 2
    )
    sc_gf = pl.kernel(
        _sc_gather_gf,
        mesh=mesh,
        out_type=[jax.ShapeDtypeStruct((FUT * B, D), f32)],
        scratch_types=list(ef_scratch),
        compiler_params=pltpu.CompilerParams(use_tc_tiling_on_sc=False),
    )
    (gf,) = sc_gf(E1, idxf)

    e1_scratch = (
        [pltpu.VMEM((NCH_H, CH_H), i32)]
        + [pltpu.VMEM((CH_H, D), f32)] * 2
        + [pltpu.SemaphoreType.DMA] * 2
    )
    sc_gh = pl.kernel(
        _sc_gather_gh,
        mesh=mesh,
        out_type=[jax.ShapeDtypeStruct((HIST * B, D), f32)],
        scratch_types=list(e1_scratch),
        compiler_params=pltpu.CompilerParams(use_tc_tiling_on_sc=False),
    )
    (gh,) = sc_gh(E1, idxh)

    sc_e0 = pl.kernel(
        _sc_gather_e0,
        mesh=mesh,
        out_type=[jax.ShapeDtypeStruct((B, 2 * D), f32)],
        scratch_types=[
            pltpu.VMEM((1, S_PER_W), i32),
            pltpu.VMEM((S_PER_W, 2 * D), f32),
            pltpu.SemaphoreType.DMA,
        ],
    )
    (s2_rows,) = sc_e0(E0.reshape(E0.shape[0] // 2, 2 * D), idx0q)

    x_t = jnp.transpose(inputs, (1, 2, 0))
    z64 = jnp.zeros((D,), f32)
    wc = jnp.stack([z64, z64, W2.reshape(D), W3.reshape(D), W4.reshape(D),
                    W5.reshape(D), W6.reshape(D), W7.reshape(D)], axis=1)
    bc = jnp.stack([z64, z64, b2, b3, b4, b5, b6, b7], axis=1)

    fut_phys = pl.pallas_call(
        _tc_fut,
        grid=(FUT // TS,),
        in_specs=[
            pl.BlockSpec((TS, NUM_INPUTS, B), lambda t: (t + HIST // TS, 0, 0)),
            pl.BlockSpec((TS * B, D), lambda t: (t, 0)),
            pl.BlockSpec((D, NUM_INPUTS), lambda t: (0, 0)),
            pl.BlockSpec((D, NUM_INPUTS), lambda t: (0, 0)),
        ],
        out_specs=pl.BlockSpec((TS, 3, D, B), lambda t: (t, 0, 0, 0)),
        out_shape=jax.ShapeDtypeStruct((FUT, 3, D, B), f32),
    )(x_t, gf, wc, bc)

    hist_phys = pl.pallas_call(
        _tc_hist,
        grid=(HIST // TS,),
        in_specs=[
            pl.BlockSpec((TS, NUM_INPUTS, B), lambda t: (t, 0, 0)),
            pl.BlockSpec((TS * B, D), lambda t: (t, 0)),
            pl.BlockSpec((D, NUM_INPUTS), lambda t: (0, 0)),
            pl.BlockSpec((D, NUM_INPUTS), lambda t: (0, 0)),
        ],
        out_specs=pl.BlockSpec((TS, 7, D, B), lambda t: (t, 0, 0, 0)),
        out_shape=jax.ShapeDtypeStruct((HIST, 7, D, B), f32),
    )(x_t, gh, wc, bc)

    stat_phys = pl.pallas_call(
        _tc_static,
        grid=(1,),
        in_specs=[
            pl.BlockSpec((B, 2 * D), lambda t: (0, 0)),
            pl.BlockSpec((B, 1), lambda t: (0, 0)),
        ],
        out_specs=pl.BlockSpec((1, D, B), lambda t: (0, 0, 0)),
        out_shape=jax.ShapeDtypeStruct((1, D, B), f32),
    )(s2_rows, par0)

    static = jnp.transpose(stat_phys, (2, 0, 1))
    historical = jnp.transpose(hist_phys, (3, 0, 1, 2))
    future = jnp.transpose(fut_phys, (3, 0, 1, 2))
    return (static, historical, future)
